# baseline (speedup 1.0000x reference)
import jax
import jax.numpy as jnp
from jax import lax
from jax.experimental import pallas as pl
from jax.experimental.pallas import tpu as pltpu
from jax.experimental.pallas import tpu_sc as plsc

VOCAB = 100000
EMB = 64
BATCH = 1024
CTX = 20

NC = 2
NS = 16
L = 16
NW = NC * NS
B_PER_W = BATCH // NW
IDX_PER_W = B_PER_W * CTX
GCHUNK = 128
NCHUNK = IDX_PER_W // GCHUNK


PAR_W = IDX_PER_W + 2 * L


def _pool_body(pairs_hbm, pidx_hbm, par_hbm, out_hbm,
               pidx_v, par_v, rows2_v, pooled_v, sem):
    wid = lax.axis_index("s") * NC + lax.axis_index("c")
    pltpu.sync_copy(pidx_hbm.at[wid], pidx_v)
    pltpu.sync_copy(par_hbm.at[wid], par_v)
    descs = [
        pltpu.async_copy(
            pairs_hbm.at[pidx_v.at[j]],
            rows2_v.at[pl.ds(j * GCHUNK, GCHUNK)],
            sem,
        )
        for j in range(NCHUNK)
    ]

    inv = jnp.float32(1.0 / CTX)

    def body(b, carry):
        accs = [jnp.zeros((L,), jnp.float32) for _ in range(2 * EMB // L)]
        for t in range(CTX):
            flat = b * CTX + t
            mhi = par_v[pl.ds(flat, L)][0]
            mlo = jnp.float32(1.0) - mhi
            for c in range(2 * EMB // L):
                m = mlo if c < EMB // L else mhi
                accs[c] = accs[c] + rows2_v[flat, pl.ds(c * L, L)] * m
        for c in range(EMB // L):
            pooled_v[b, pl.ds(c * L, L)] = (accs[c] + accs[c + EMB // L]) * inv
        return carry

    b_done = 0
    for j in range(NCHUNK):
        descs[j].wait()
        b_next = min(B_PER_W, ((j + 1) * GCHUNK) // CTX)
        lax.fori_loop(b_done, b_next, body, 0)
        b_done = b_next
    pltpu.sync_copy(pooled_v, out_hbm.at[pl.ds(wid * B_PER_W, B_PER_W)])


def _pool(pairs, pidx3, par3):
    mesh = plsc.VectorSubcoreMesh(
        core_axis_name="c", subcore_axis_name="s",
        num_cores=NC, num_subcores=NS,
    )
    return pl.kernel(
        _pool_body,
        out_type=jax.ShapeDtypeStruct((BATCH, EMB), jnp.float32),
        mesh=mesh,
        compiler_params=pltpu.CompilerParams(use_tc_tiling_on_sc=True),
        scratch_types=[
            pltpu.VMEM((NCHUNK, GCHUNK), jnp.int32),
            pltpu.VMEM((PAR_W,), jnp.float32),
            pltpu.VMEM((IDX_PER_W, 2 * EMB), jnp.float32),
            pltpu.VMEM((B_PER_W, EMB), jnp.float32),
            pltpu.SemaphoreType.DMA,
        ],
    )(pairs, pidx3, par3)


SPLIT = 50176
T_BLK = 1792
TGRID = SPLIT // T_BLK


def _pairs_body(lo_ref, hi_ref, o_ref):
    o_ref[...] = jnp.concatenate(
        [lo_ref[...].T, hi_ref[...].T], axis=1)


def _make_pairs(emb_table):
    return pl.pallas_call(
        _pairs_body,
        grid=(TGRID,),
        in_specs=[
            pl.BlockSpec((EMB, T_BLK), lambda i: (0, i)),
            pl.BlockSpec((EMB, T_BLK), lambda i: (0, i + TGRID)),
        ],
        out_specs=pl.BlockSpec((T_BLK, 2 * EMB), lambda i: (i, 0)),
        out_shape=jax.ShapeDtypeStruct((SPLIT, 2 * EMB), jnp.float32),
    )(emb_table.T, emb_table.T)


M_BLK = 4096
NGRID = (VOCAB + M_BLK - 1) // M_BLK


def _matmul_body(wt_ref, xt_ref, o_ref):
    o_ref[...] = lax.dot_general(
        wt_ref[...], xt_ref[...], (((0,), (0,)), ((), ())),
        preferred_element_type=jnp.float32,
    )


def _project(pooled, W):
    out_t = pl.pallas_call(
        _matmul_body,
        grid=(NGRID,),
        in_specs=[
            pl.BlockSpec((EMB, M_BLK), lambda i: (0, i)),
            pl.BlockSpec((EMB, BATCH), lambda i: (0, 0)),
        ],
        out_specs=pl.BlockSpec((M_BLK, BATCH), lambda i: (i, 0)),
        out_shape=jax.ShapeDtypeStruct((VOCAB, BATCH), jnp.float32),
    )(W.T, pooled.T)
    return out_t.T


def kernel(inpt, emb_table, W):
    idx = inpt.astype(jnp.int32)
    hi = idx >= SPLIT
    pidx3 = jnp.where(hi, idx - SPLIT, idx).reshape(NW, NCHUNK, GCHUNK)
    par = hi.astype(jnp.float32).reshape(NW, IDX_PER_W)
    par_pad = jnp.pad(par, ((0, 0), (0, PAR_W - IDX_PER_W)))
    pairs = _make_pairs(emb_table)
    pooled = _pool(pairs, pidx3, par_pad)
    return _project(pooled, W)


# speedup vs baseline: 1.0290x; 1.0290x over previous
import jax
import jax.numpy as jnp
from jax import lax
from jax.experimental import pallas as pl
from jax.experimental.pallas import tpu as pltpu
from jax.experimental.pallas import tpu_sc as plsc

VOCAB = 100000
EMB = 64
BATCH = 1024
CTX = 20

NC = 2
NS = 16
L = 16
NW = NC * NS
B_PER_W = BATCH // NW
IDX_PER_W = B_PER_W * CTX
GCHUNK = 128
NCHUNK = IDX_PER_W // GCHUNK


PAR_W = IDX_PER_W + 2 * L


def _pool_body(pairs_hbm, pidx_hbm, par_hbm, out_hbm,
               pidx_v, par_v, rows2_v, pooled_v, sem):
    wid = lax.axis_index("s") * NC + lax.axis_index("c")
    pltpu.sync_copy(pidx_hbm.at[wid], pidx_v)
    pltpu.sync_copy(par_hbm.at[wid], par_v)
    descs = [
        pltpu.async_copy(
            pairs_hbm.at[pidx_v.at[j]],
            rows2_v.at[pl.ds(j * GCHUNK, GCHUNK)],
            sem,
        )
        for j in range(NCHUNK)
    ]

    inv = jnp.float32(1.0 / CTX)

    def body(b, carry):
        accs = [jnp.zeros((L,), jnp.float32) for _ in range(2 * EMB // L)]
        for t in range(CTX):
            flat = b * CTX + t
            mhi = par_v[pl.ds(flat, L)][0]
            mlo = jnp.float32(1.0) - mhi
            for c in range(2 * EMB // L):
                m = mlo if c < EMB // L else mhi
                accs[c] = accs[c] + rows2_v[flat, pl.ds(c * L, L)] * m
        for c in range(EMB // L):
            pooled_v[b, pl.ds(c * L, L)] = (accs[c] + accs[c + EMB // L]) * inv
        return carry

    b_done = 0
    for j in range(NCHUNK):
        descs[j].wait()
        b_next = min(B_PER_W, ((j + 1) * GCHUNK) // CTX)
        lax.fori_loop(b_done, b_next, body, 0)
        b_done = b_next
    pltpu.sync_copy(pooled_v, out_hbm.at[pl.ds(wid * B_PER_W, B_PER_W)])


def _pool(pairs, pidx3, par3):
    mesh = plsc.VectorSubcoreMesh(
        core_axis_name="c", subcore_axis_name="s",
        num_cores=NC, num_subcores=NS,
    )
    return pl.kernel(
        _pool_body,
        out_type=jax.ShapeDtypeStruct((BATCH, EMB), jnp.float32),
        mesh=mesh,
        compiler_params=pltpu.CompilerParams(use_tc_tiling_on_sc=True),
        scratch_types=[
            pltpu.VMEM((NCHUNK, GCHUNK), jnp.int32),
            pltpu.VMEM((PAR_W,), jnp.float32),
            pltpu.VMEM((IDX_PER_W, 2 * EMB), jnp.float32),
            pltpu.VMEM((B_PER_W, EMB), jnp.float32),
            pltpu.SemaphoreType.DMA,
        ],
    )(pairs, pidx3, par3)


SPLIT = 50048
T_BLK = 2944
TGRID = SPLIT // T_BLK


def _pairs_body(lo_ref, hi_ref, o_ref):
    o_ref[...] = jnp.concatenate(
        [lo_ref[...].T, hi_ref[...].T], axis=1)


def _make_pairs(emb_table):
    return pl.pallas_call(
        _pairs_body,
        grid=(TGRID,),
        in_specs=[
            pl.BlockSpec((EMB, T_BLK), lambda i: (0, i)),
            pl.BlockSpec((EMB, T_BLK), lambda i: (0, i + TGRID)),
        ],
        out_specs=pl.BlockSpec((T_BLK, 2 * EMB), lambda i: (i, 0)),
        out_shape=jax.ShapeDtypeStruct((SPLIT, 2 * EMB), jnp.float32),
    )(emb_table.T, emb_table.T)


M_BLK = 4096
NGRID = (VOCAB + M_BLK - 1) // M_BLK


def _matmul_body(wt_ref, xt_ref, o_ref):
    o_ref[...] = lax.dot_general(
        wt_ref[...], xt_ref[...], (((0,), (0,)), ((), ())),
        preferred_element_type=jnp.float32,
    )


def _project(pooled, W):
    out_t = pl.pallas_call(
        _matmul_body,
        grid=(NGRID,),
        in_specs=[
            pl.BlockSpec((EMB, M_BLK), lambda i: (0, i)),
            pl.BlockSpec((EMB, BATCH), lambda i: (0, 0)),
        ],
        out_specs=pl.BlockSpec((M_BLK, BATCH), lambda i: (i, 0)),
        out_shape=jax.ShapeDtypeStruct((VOCAB, BATCH), jnp.float32),
    )(W.T, pooled.T)
    return out_t.T


def kernel(inpt, emb_table, W):
    idx = inpt.astype(jnp.int32)
    hi = idx >= SPLIT
    pidx3 = jnp.where(hi, idx - SPLIT, idx).reshape(NW, NCHUNK, GCHUNK)
    par = hi.astype(jnp.float32).reshape(NW, IDX_PER_W)
    par_pad = jnp.pad(par, ((0, 0), (0, PAR_W - IDX_PER_W)))
    pairs = _make_pairs(emb_table)
    pooled = _pool(pairs, pidx3, par_pad)
    return _project(pooled, W)
